# baseline (speedup 1.0000x reference)
import functools

import jax
import jax.numpy as jnp
from jax import lax
from jax.experimental import pallas as pl
from jax.experimental.pallas import tpu as pltpu
from jax.experimental.pallas import tpu_sc as plsc

N = 10000
D_IN = 128
HID = 128
C = 9
CP = 16
N_MID = 3

NC = 2
NS = 16
NPAD = 10240
RPS = NPAD // NS
E = 320000
E2 = E + N
EPAD = 164 * 2048
EPT = EPAD // NS
NCH = EPT // 128
FH = 64
NT = NC * NS

BLK = 640
NBLK = NPAD // BLK


def _make_edge_kernel():
  mesh = plsc.VectorSubcoreMesh(core_axis_name="c", subcore_axis_name="s")

  @functools.partial(
      pl.kernel,
      mesh=mesh,
      name="gat_edge_phase",
      compiler_params=pltpu.CompilerParams(use_tc_tiling_on_sc=False,
                                           needs_layout_passes=False),
      out_type=[
          jax.ShapeDtypeStruct((NC, NPAD, FH), jnp.float32),
          jax.ShapeDtypeStruct((NT, NPAD), jnp.float32),
      ],
      scratch_types=[
          pltpu.VMEM((2 * NPAD,), jnp.float32),
          pltpu.VMEM((2, 128), jnp.int32),
          pltpu.VMEM((NCH, 128), jnp.int32),
          pltpu.VMEM((128,), jnp.float32),
          pltpu.VMEM((NPAD,), jnp.float32),
          pltpu.VMEM((2, 128, FH), jnp.float32),
          pltpu.VMEM((2, 128, FH), jnp.float32),
          pltpu.VMEM_SHARED((NPAD, FH), jnp.float32),
          pltpu.SemaphoreType.DMA,
          pltpu.SemaphoreType.DMA,
          pltpu.SemaphoreType.DMA,
          pltpu.SemaphoreType.DMA,
          pltpu.SemaphoreType.DMA,
          pltpu.SemaphoreType.DMA,
      ],
  )
  def edge_kernel(h_hbm, aa_hbm, src_hbm, dst_hbm, num_out, den_out,
                  aa_v, src_r, dst_v, ee_c, den_v, rows_v, wrows_v, acc,
                  sem_g0, sem_g1, sem_s0, sem_s1, sem_i0, sem_i1):
    c = lax.axis_index("c")
    s = lax.axis_index("s")
    wid = s * NC + c

    sem_g = (sem_g0, sem_g1)
    sem_s = (sem_s0, sem_s1)
    sem_i = (sem_i0, sem_i1)

    pltpu.async_copy(src_hbm.at[s, 0], src_r.at[0], sem_i0)
    pltpu.async_copy(src_hbm.at[s, 1], src_r.at[1], sem_i1)
    pltpu.sync_copy(aa_hbm, aa_v)
    pltpu.sync_copy(dst_hbm.at[s], dst_v)

    zero16 = jnp.zeros((16,), jnp.float32)

    @pl.loop(0, NPAD // 16)
    def _(r):
      den_v[pl.ds(r * 16, 16)] = zero16

    @pl.loop(0, 128)
    def _(r):
      for dd in range(FH // 16):
        wrows_v[0, r, pl.ds(dd * 16, 16)] = zero16

    for r0 in range(8):
      pltpu.sync_copy(wrows_v.at[0, pl.ds(0, 80)],
                      acc.at[pl.ds(s * RPS + r0 * 80, 80)])
    plsc.subcore_barrier()

    def prescale(b):
      for kk in range(8):
        sl16 = pl.ds(kk * 16, 16)
        src_r[b, sl16] = src_r[b, sl16] * 2 + c

    pltpu.make_async_copy(src_hbm.at[s, 0], src_r.at[0], sem_i0).wait()
    prescale(0)
    pltpu.async_copy(h_hbm.at[src_r.at[0]], rows_v.at[0], sem_g0)

    @pl.loop(0, NCH, step=2)
    def _(g):
      for b in range(2):
        gb = g + b
        nxt = jnp.minimum(gb + 1, NCH - 1)
        pltpu.make_async_copy(src_hbm.at[s, nxt], src_r.at[1 - b],
                              sem_i[1 - b]).wait()
        prescale(1 - b)
        pltpu.async_copy(h_hbm.at[src_r.at[1 - b]], rows_v.at[1 - b],
                         sem_g[1 - b])

        pltpu.make_async_copy(h_hbm.at[src_r.at[b]], rows_v.at[b],
                              sem_g[b]).wait()

        for kk in range(8):
          sl16 = pl.ds(kk * 16, 16)
          sv2 = src_r[b, sl16]
          dv = dst_v[gb, sl16]
          e = (plsc.load_gather(aa_v, [sv2 - c])
               + plsc.load_gather(aa_v, [dv * 2 + 1]))
          ee = jnp.exp(jnp.maximum(e, 0.2 * e))
          ee_c[sl16] = ee
          plsc.addupdate_scatter(den_v, [dv], ee)

        @pl.when(gb >= 2)
        def _():
          pltpu.make_async_copy(wrows_v.at[b], acc.at[dst_v.at[gb]],
                                sem_s[b]).wait()

        @pl.loop(0, 128, step=4)
        def _(j):
          for u in range(4):
            eev = plsc.load_gather(
                ee_c, [jnp.full((16,), 0, jnp.int32) + (j + u)])
            for dd in range(FH // 16):
              wrows_v[b, j + u, pl.ds(dd * 16, 16)] = (
                  rows_v[b, j + u, pl.ds(dd * 16, 16)] * eev)

        pltpu.async_copy(wrows_v.at[b], acc.at[dst_v.at[gb]], sem_s[b],
                         add=True)
        nxt2 = jnp.minimum(gb + 2, NCH - 1)
        pltpu.async_copy(src_hbm.at[s, nxt2], src_r.at[b], sem_i[b])

    pltpu.make_async_copy(h_hbm.at[src_r.at[0]], rows_v.at[0],
                          sem_g0).wait()
    pltpu.make_async_copy(src_hbm.at[s, NCH - 1], src_r.at[1],
                          sem_i1).wait()
    pltpu.make_async_copy(wrows_v.at[0], acc.at[dst_v.at[NCH - 2]],
                          sem_s0).wait()
    pltpu.make_async_copy(wrows_v.at[1], acc.at[dst_v.at[NCH - 1]],
                          sem_s1).wait()

    pltpu.sync_copy(den_v, den_out.at[wid])
    plsc.subcore_barrier()

    for r0 in range(8):
      off = s * RPS + r0 * 80
      pltpu.sync_copy(acc.at[pl.ds(off, 80)], num_out.at[c, pl.ds(off, 80)])

  return edge_kernel


_edge128 = _make_edge_kernel()


def _rowmask(i, blk):
  row = i * blk + lax.broadcasted_iota(jnp.int32, (blk, 1), 0)
  return row < N


def _pre_body(x_ref, w_ref, a_ref, h_out, aa_out):
  h = jnp.dot(x_ref[...], w_ref[...], preferred_element_type=jnp.float32)
  h_out[...] = h
  aa_out[...] = jnp.dot(h, a_ref[...], preferred_element_type=jnp.float32)


_pre = pl.pallas_call(
    _pre_body,
    grid=(NBLK,),
    in_specs=[
        pl.BlockSpec((BLK, D_IN), lambda i: (i, 0)),
        pl.BlockSpec((D_IN, HID), lambda i: (0, 0)),
        pl.BlockSpec((HID, 2), lambda i: (0, 0)),
    ],
    out_specs=[
        pl.BlockSpec((BLK, HID), lambda i: (i, 0)),
        pl.BlockSpec((BLK, 2), lambda i: (i, 0)),
    ],
    out_shape=[
        jax.ShapeDtypeStruct((NPAD, HID), jnp.float32),
        jax.ShapeDtypeStruct((NPAD, 2), jnp.float32),
    ],
)


def _gat_from_acc(num_ref, den_ref, b):
  numer = jnp.concatenate([num_ref[0], num_ref[1]], axis=1)
  den = jnp.sum(den_ref[...].T, axis=1, keepdims=True) * 0.5
  return numer / (den + 1e-16) + b


def _pass1_body(num_ref, den_ref, b_ref, res_ref, fl_ref, x_out, part_out):
  i = pl.program_id(0)
  gat = _gat_from_acc(num_ref, den_ref, b_ref[...])
  gat = gat + res_ref[...] * fl_ref[0, 0]
  x = jnp.where(_rowmask(i, BLK), gat, 0.0)
  x_out[...] = x
  part_out[0, 0] = jnp.sum(x, axis=0)
  part_out[0, 1] = jnp.sum(x * x, axis=0)


_pass1 = pl.pallas_call(
    _pass1_body,
    grid=(NBLK,),
    in_specs=[
        pl.BlockSpec((NC, BLK, FH), lambda i: (0, i, 0)),
        pl.BlockSpec((NT, BLK), lambda i: (0, i)),
        pl.BlockSpec((1, HID), lambda i: (0, 0)),
        pl.BlockSpec((BLK, HID), lambda i: (i, 0)),
        pl.BlockSpec((1, 1), lambda i: (0, 0)),
    ],
    out_specs=[
        pl.BlockSpec((BLK, HID), lambda i: (i, 0)),
        pl.BlockSpec((1, 2, HID), lambda i: (i, 0, 0)),
    ],
    out_shape=[
        jax.ShapeDtypeStruct((NPAD, HID), jnp.float32),
        jax.ShapeDtypeStruct((NBLK, 2, HID), jnp.float32),
    ],
)


def _make_pass2(dout):
  def body(x_ref, part_ref, g_ref, bb_ref, w_ref, a_ref,
           h_out, aa_out, hn_out):
    i = pl.program_id(0)
    m = jnp.sum(part_ref[:, 0, :], axis=0) / N
    v = jnp.sum(part_ref[:, 1, :], axis=0) / N - m * m
    x = x_ref[...]
    xn = g_ref[...] * (x - m[None, :]) / jnp.sqrt(v + 1e-5)[None, :] + bb_ref[...]
    hn = jnp.maximum(xn, 0.1 * xn)
    hn = jnp.where(_rowmask(i, BLK), hn, 0.0)
    hn_out[...] = hn
    h2 = jnp.dot(hn, w_ref[...], preferred_element_type=jnp.float32)
    h_out[...] = h2
    aa_out[...] = jnp.dot(h2, a_ref[...], preferred_element_type=jnp.float32)

  return pl.pallas_call(
      body,
      grid=(NBLK,),
      in_specs=[
          pl.BlockSpec((BLK, HID), lambda i: (i, 0)),
          pl.BlockSpec((NBLK, 2, HID), lambda i: (0, 0, 0)),
          pl.BlockSpec((1, HID), lambda i: (0, 0)),
          pl.BlockSpec((1, HID), lambda i: (0, 0)),
          pl.BlockSpec((HID, dout), lambda i: (0, 0)),
          pl.BlockSpec((dout, 2), lambda i: (0, 0)),
      ],
      out_specs=[
          pl.BlockSpec((BLK, dout), lambda i: (i, 0)),
          pl.BlockSpec((BLK, 2), lambda i: (i, 0)),
          pl.BlockSpec((BLK, HID), lambda i: (i, 0)),
      ],
      out_shape=[
          jax.ShapeDtypeStruct((NPAD, dout), jnp.float32),
          jax.ShapeDtypeStruct((NPAD, 2), jnp.float32),
          jax.ShapeDtypeStruct((NPAD, HID), jnp.float32),
      ],
  )


_pass2_128 = _make_pass2(HID)


def _final_body(x_ref, out_ref):
  g = x_ref[...]
  out_ref[...] = jnp.maximum(g, 0.1 * g)[:, :C]


_final = pl.pallas_call(
    _final_body,
    grid=(10,),
    in_specs=[pl.BlockSpec((1000, HID), lambda i: (i, 0))],
    out_specs=pl.BlockSpec((1000, C), lambda i: (i, 0)),
    out_shape=jax.ShapeDtypeStruct((N, C), jnp.float32),
)


def kernel(x, edge_index, edge_type, edge_distance, params):
  del edge_type, edge_distance
  sl = jnp.arange(N, dtype=jnp.int32)
  padv = jnp.full((EPAD - E2,), N, jnp.int32)
  src = jnp.concatenate([edge_index[0].astype(jnp.int32), sl, padv])
  dst = jnp.concatenate([edge_index[1].astype(jnp.int32), sl, padv])
  srcs = src.reshape(NS, NCH, 128)
  dsts = dst.reshape(NS, NCH, 128)

  xp = jnp.pad(x, ((0, NPAD - N), (0, 0)))

  def att(l):
    return jnp.stack([params[f"att_src{l}"], params[f"att_dst{l}"]], axis=1)

  def edge(h, aa):
    return _edge128(h.reshape(2 * NPAD, FH), aa.reshape(-1), srcs, dsts)

  h, aa = _pre(xp, params["W0"], att(0))

  lf = N_MID + 1
  w4 = jnp.pad(params[f"W{lf}"], ((0, 0), (0, HID - C)))
  a4 = jnp.pad(att(lf), ((0, HID - C), (0, 0)))
  b4 = jnp.pad(params[f"b{lf}"], (0, HID - C))
  zvec = jnp.zeros((HID,), jnp.float32)
  bs = jnp.stack([params["b0"], params["b1"], params["b2"], params["b3"], b4]
                 )[:, None, :]
  bngs = jnp.stack([params["bn_g1"], params["bn_g2"], params["bn_g3"],
                    params["bn_g4"], jnp.ones((HID,), jnp.float32)])[:, None, :]
  bnbs = jnp.stack([params["bn_b1"], params["bn_b2"], params["bn_b3"],
                    params["bn_b4"], zvec])[:, None, :]
  ws = jnp.stack([params["W1"], params["W2"], params["W3"], w4,
                  jnp.zeros((HID, HID), jnp.float32)])
  aas = jnp.stack([att(1), att(2), att(3), a4, jnp.zeros((HID, 2), jnp.float32)])
  flags = jnp.array([0, 1, 1, 1, 0], jnp.float32).reshape(5, 1, 1)

  def body(carry, xs):
    hh, av, res, _ = carry
    b, g_, bb, w, a, fl = xs
    num, den = edge(hh, av)
    xt, part = _pass1(num, den, b, res, fl)
    h2, aa2, hn = _pass2_128(xt, part, g_, bb, w, a)
    return (h2, aa2, hn, xt), None

  zfeat = jnp.zeros((NPAD, HID), jnp.float32)
  (_, _, _, xt), _ = lax.scan(
      body, (h, aa, zfeat, zfeat), (bs, bngs, bnbs, ws, aas, flags))
  return _final(xt)


# speedup vs baseline: 1.5695x; 1.5695x over previous
import functools

import jax
import jax.numpy as jnp
from jax import lax
from jax.experimental import pallas as pl
from jax.experimental.pallas import tpu as pltpu
from jax.experimental.pallas import tpu_sc as plsc

N = 10000
D_IN = 128
HID = 128
C = 9
CP = 16
N_MID = 3

NC = 2
NS = 16
NPAD = 10240
RPS = NPAD // NS
E = 320000
E2 = E + N
EPAD = 164 * 2048
EPT = EPAD // NS
NCH = EPT // 128
FH = 64
NT = NC * NS

BLK = 640
NBLK = NPAD // BLK


def _make_edge_kernel():
  mesh = plsc.VectorSubcoreMesh(core_axis_name="c", subcore_axis_name="s")

  @functools.partial(
      pl.kernel,
      mesh=mesh,
      name="gat_edge_phase",
      compiler_params=pltpu.CompilerParams(use_tc_tiling_on_sc=False,
                                           needs_layout_passes=False),
      out_type=[
          jax.ShapeDtypeStruct((NC, NPAD, FH), jnp.float32),
          jax.ShapeDtypeStruct((NT, NPAD), jnp.float32),
      ],
      scratch_types=[
          pltpu.VMEM((2 * NPAD,), jnp.float32),
          pltpu.VMEM((2, 128), jnp.int32),
          pltpu.VMEM((2, 128), jnp.int32),
          pltpu.VMEM((128,), jnp.float32),
          pltpu.VMEM((NPAD,), jnp.float32),
          pltpu.VMEM((2, 128, FH), jnp.float32),
          pltpu.VMEM_SHARED((NPAD, FH), jnp.float32),
          pltpu.VMEM_SHARED((NPAD, FH), jnp.float32),
          pltpu.SemaphoreType.DMA,
          pltpu.SemaphoreType.DMA,
          pltpu.SemaphoreType.DMA,
          pltpu.SemaphoreType.DMA,
          pltpu.SemaphoreType.DMA,
          pltpu.SemaphoreType.DMA,
      ],
  )
  def edge_kernel(h_hbm, aa_hbm, src_hbm, dst_hbm, num_out, den_out,
                  aa_v, src_r, dst_r, ee_c, den_v, rows_v, h_spm, acc,
                  sem_g0, sem_g1, sem_i0, sem_i1, sem_j0, sem_j1):
    c = lax.axis_index("c")
    s = lax.axis_index("s")
    wid = s * NC + c

    sem_g = (sem_g0, sem_g1)
    sem_i = (sem_i0, sem_i1)
    sem_j = (sem_j0, sem_j1)

    pltpu.async_copy(src_hbm.at[s, 0], src_r.at[0], sem_i0)
    pltpu.async_copy(src_hbm.at[s, 1], src_r.at[1], sem_i1)
    pltpu.async_copy(dst_hbm.at[s, 0], dst_r.at[0], sem_j0)
    pltpu.async_copy(dst_hbm.at[s, 1], dst_r.at[1], sem_j1)
    pltpu.sync_copy(aa_hbm, aa_v)
    pltpu.sync_copy(h_hbm.at[c, pl.ds(s * RPS, RPS)],
                    h_spm.at[pl.ds(s * RPS, RPS)])

    zero16 = jnp.zeros((16,), jnp.float32)

    @pl.loop(0, NPAD // 16)
    def _(r):
      den_v[pl.ds(r * 16, 16)] = zero16

    @pl.loop(0, 128)
    def _(r):
      for dd in range(FH // 16):
        rows_v[0, r, pl.ds(dd * 16, 16)] = zero16

    for r0 in range(8):
      pltpu.sync_copy(rows_v.at[0, pl.ds(0, 80)],
                      acc.at[pl.ds(s * RPS + r0 * 80, 80)])
    plsc.subcore_barrier()

    def process(b):
      for kk in range(8):
        sl16 = pl.ds(kk * 16, 16)
        sv = src_r[b, sl16]
        dv = dst_r[b, sl16]
        e = (plsc.load_gather(aa_v, [sv * 2])
             + plsc.load_gather(aa_v, [dv * 2 + 1]))
        ee = jnp.exp(jnp.maximum(e, 0.2 * e))
        ee_c[sl16] = ee
        plsc.addupdate_scatter(den_v, [dv], ee)

      @pl.loop(0, 128, step=4)
      def _(j):
        for u in range(4):
          eev = plsc.load_gather(ee_c, [jnp.full((16,), 0, jnp.int32) + (j + u)])
          for dd in range(FH // 16):
            rows_v[b, j + u, pl.ds(dd * 16, 16)] = (
                rows_v[b, j + u, pl.ds(dd * 16, 16)] * eev)

      pltpu.sync_copy(rows_v.at[b], acc.at[dst_r.at[b]], add=True)

    pltpu.make_async_copy(src_hbm.at[s, 0], src_r.at[0], sem_i0).wait()
    pltpu.async_copy(h_spm.at[src_r.at[0]], rows_v.at[0], sem_g0)

    @pl.loop(0, NCH, step=2)
    def _(g):
      for b in range(2):
        gb = g + b
        nxt = jnp.minimum(gb + 1, NCH - 1)
        pltpu.make_async_copy(src_hbm.at[s, nxt], src_r.at[1 - b],
                              sem_i[1 - b]).wait()
        pltpu.async_copy(h_spm.at[src_r.at[1 - b]], rows_v.at[1 - b],
                         sem_g[1 - b])
        pltpu.make_async_copy(dst_hbm.at[s, nxt], dst_r.at[b],
                              sem_j[b]).wait()
        pltpu.make_async_copy(h_spm.at[src_r.at[b]], rows_v.at[b],
                              sem_g[b]).wait()
        process(b)
        nxt2 = jnp.minimum(gb + 2, NCH - 1)
        pltpu.async_copy(src_hbm.at[s, nxt2], src_r.at[b], sem_i[b])
        pltpu.async_copy(dst_hbm.at[s, nxt2], dst_r.at[b], sem_j[b])

    pltpu.make_async_copy(h_spm.at[src_r.at[0]], rows_v.at[0],
                          sem_g0).wait()
    pltpu.make_async_copy(src_hbm.at[s, NCH - 1], src_r.at[1],
                          sem_i1).wait()
    pltpu.make_async_copy(dst_hbm.at[s, NCH - 1], dst_r.at[0],
                          sem_j0).wait()
    pltpu.make_async_copy(dst_hbm.at[s, NCH - 1], dst_r.at[1],
                          sem_j1).wait()

    pltpu.sync_copy(den_v, den_out.at[wid])
    plsc.subcore_barrier()

    for r0 in range(8):
      off = s * RPS + r0 * 80
      pltpu.sync_copy(acc.at[pl.ds(off, 80)], num_out.at[c, pl.ds(off, 80)])

  return edge_kernel


_edge128 = _make_edge_kernel()


def _rowmask(i, blk):
  row = i * blk + lax.broadcasted_iota(jnp.int32, (blk, 1), 0)
  return row < N


def _pre_body(x_ref, w_ref, a_ref, h_out, aa_out):
  h = jnp.dot(x_ref[...], w_ref[...], preferred_element_type=jnp.float32)
  h_out[0] = h[:, :FH]
  h_out[1] = h[:, FH:]
  aa_out[...] = jnp.dot(h, a_ref[...], preferred_element_type=jnp.float32)


_pre = pl.pallas_call(
    _pre_body,
    grid=(NBLK,),
    in_specs=[
        pl.BlockSpec((BLK, D_IN), lambda i: (i, 0)),
        pl.BlockSpec((D_IN, HID), lambda i: (0, 0)),
        pl.BlockSpec((HID, 2), lambda i: (0, 0)),
    ],
    out_specs=[
        pl.BlockSpec((NC, BLK, FH), lambda i: (0, i, 0)),
        pl.BlockSpec((BLK, 2), lambda i: (i, 0)),
    ],
    out_shape=[
        jax.ShapeDtypeStruct((NC, NPAD, FH), jnp.float32),
        jax.ShapeDtypeStruct((NPAD, 2), jnp.float32),
    ],
)


def _gat_from_acc(num_ref, den_ref, b):
  numer = jnp.concatenate([num_ref[0], num_ref[1]], axis=1)
  den = jnp.sum(den_ref[...].T, axis=1, keepdims=True) * 0.5
  return numer / (den + 1e-16) + b


def _pass1_body(num_ref, den_ref, b_ref, res_ref, fl_ref, x_out, part_out):
  i = pl.program_id(0)
  gat = _gat_from_acc(num_ref, den_ref, b_ref[...])
  gat = gat + res_ref[...] * fl_ref[0, 0]
  x = jnp.where(_rowmask(i, BLK), gat, 0.0)
  x_out[...] = x
  part_out[0, 0] = jnp.sum(x, axis=0)
  part_out[0, 1] = jnp.sum(x * x, axis=0)


_pass1 = pl.pallas_call(
    _pass1_body,
    grid=(NBLK,),
    in_specs=[
        pl.BlockSpec((NC, BLK, FH), lambda i: (0, i, 0)),
        pl.BlockSpec((NT, BLK), lambda i: (0, i)),
        pl.BlockSpec((1, HID), lambda i: (0, 0)),
        pl.BlockSpec((BLK, HID), lambda i: (i, 0)),
        pl.BlockSpec((1, 1), lambda i: (0, 0)),
    ],
    out_specs=[
        pl.BlockSpec((BLK, HID), lambda i: (i, 0)),
        pl.BlockSpec((1, 2, HID), lambda i: (i, 0, 0)),
    ],
    out_shape=[
        jax.ShapeDtypeStruct((NPAD, HID), jnp.float32),
        jax.ShapeDtypeStruct((NBLK, 2, HID), jnp.float32),
    ],
)


def _make_pass2(dout):
  def body(x_ref, part_ref, g_ref, bb_ref, w_ref, a_ref,
           h_out, aa_out, hn_out):
    i = pl.program_id(0)
    m = jnp.sum(part_ref[:, 0, :], axis=0) / N
    v = jnp.sum(part_ref[:, 1, :], axis=0) / N - m * m
    x = x_ref[...]
    xn = g_ref[...] * (x - m[None, :]) / jnp.sqrt(v + 1e-5)[None, :] + bb_ref[...]
    hn = jnp.maximum(xn, 0.1 * xn)
    hn = jnp.where(_rowmask(i, BLK), hn, 0.0)
    hn_out[...] = hn
    h2 = jnp.dot(hn, w_ref[...], preferred_element_type=jnp.float32)
    h_out[0] = h2[:, :FH]
    h_out[1] = h2[:, FH:]
    aa_out[...] = jnp.dot(h2, a_ref[...], preferred_element_type=jnp.float32)

  return pl.pallas_call(
      body,
      grid=(NBLK,),
      in_specs=[
          pl.BlockSpec((BLK, HID), lambda i: (i, 0)),
          pl.BlockSpec((NBLK, 2, HID), lambda i: (0, 0, 0)),
          pl.BlockSpec((1, HID), lambda i: (0, 0)),
          pl.BlockSpec((1, HID), lambda i: (0, 0)),
          pl.BlockSpec((HID, dout), lambda i: (0, 0)),
          pl.BlockSpec((dout, 2), lambda i: (0, 0)),
      ],
      out_specs=[
          pl.BlockSpec((NC, BLK, dout // 2), lambda i: (0, i, 0)),
          pl.BlockSpec((BLK, 2), lambda i: (i, 0)),
          pl.BlockSpec((BLK, HID), lambda i: (i, 0)),
      ],
      out_shape=[
          jax.ShapeDtypeStruct((NC, NPAD, dout // 2), jnp.float32),
          jax.ShapeDtypeStruct((NPAD, 2), jnp.float32),
          jax.ShapeDtypeStruct((NPAD, HID), jnp.float32),
      ],
  )


_pass2_128 = _make_pass2(HID)


def _final_body(x_ref, out_ref):
  g = x_ref[...]
  out_ref[...] = jnp.maximum(g, 0.1 * g)[:, :C]


_final = pl.pallas_call(
    _final_body,
    grid=(10,),
    in_specs=[pl.BlockSpec((1000, HID), lambda i: (i, 0))],
    out_specs=pl.BlockSpec((1000, C), lambda i: (i, 0)),
    out_shape=jax.ShapeDtypeStruct((N, C), jnp.float32),
)


def kernel(x, edge_index, edge_type, edge_distance, params):
  del edge_type, edge_distance
  sl = jnp.arange(N, dtype=jnp.int32)
  padv = jnp.full((EPAD - E2,), N, jnp.int32)
  src = jnp.concatenate([edge_index[0].astype(jnp.int32), sl, padv])
  dst = jnp.concatenate([edge_index[1].astype(jnp.int32), sl, padv])
  srcs = src.reshape(NS, NCH, 128)
  dsts = dst.reshape(NS, NCH, 128)

  xp = jnp.pad(x, ((0, NPAD - N), (0, 0)))

  def att(l):
    return jnp.stack([params[f"att_src{l}"], params[f"att_dst{l}"]], axis=1)

  def edge(h, aa):
    return _edge128(h, aa.reshape(-1), srcs, dsts)

  h, aa = _pre(xp, params["W0"], att(0))

  lf = N_MID + 1
  w4 = jnp.pad(params[f"W{lf}"], ((0, 0), (0, HID - C)))
  a4 = jnp.pad(att(lf), ((0, HID - C), (0, 0)))
  b4 = jnp.pad(params[f"b{lf}"], (0, HID - C))
  zvec = jnp.zeros((HID,), jnp.float32)
  bs = jnp.stack([params["b0"], params["b1"], params["b2"], params["b3"], b4]
                 )[:, None, :]
  bngs = jnp.stack([params["bn_g1"], params["bn_g2"], params["bn_g3"],
                    params["bn_g4"], jnp.ones((HID,), jnp.float32)])[:, None, :]
  bnbs = jnp.stack([params["bn_b1"], params["bn_b2"], params["bn_b3"],
                    params["bn_b4"], zvec])[:, None, :]
  ws = jnp.stack([params["W1"], params["W2"], params["W3"], w4,
                  jnp.zeros((HID, HID), jnp.float32)])
  aas = jnp.stack([att(1), att(2), att(3), a4, jnp.zeros((HID, 2), jnp.float32)])
  flags = jnp.array([0, 1, 1, 1, 0], jnp.float32).reshape(5, 1, 1)

  def body(carry, xs):
    hh, av, res, _ = carry
    b, g_, bb, w, a, fl = xs
    num, den = edge(hh, av)
    xt, part = _pass1(num, den, b, res, fl)
    h2, aa2, hn = _pass2_128(xt, part, g_, bb, w, a)
    return (h2, aa2, hn, xt), None

  zfeat = jnp.zeros((NPAD, HID), jnp.float32)
  (_, _, _, xt), _ = lax.scan(
      body, (h, aa, zfeat, zfeat), (bs, bngs, bnbs, ws, aas, flags))
  return _final(xt)


# speedup vs baseline: 1.9744x; 1.2579x over previous
import functools

import jax
import jax.numpy as jnp
from jax import lax
from jax.experimental import pallas as pl
from jax.experimental.pallas import tpu as pltpu
from jax.experimental.pallas import tpu_sc as plsc

N = 10000
D_IN = 128
HID = 128
C = 9
CP = 16
N_MID = 3

NC = 2
NS = 16
NPAD = 10240
RPS = NPAD // NS
E = 320000
E2 = E + N
EPAD = 164 * 2048
EPT = EPAD // NS
NCH = EPT // 128
FH = 64
NT = NC * NS

BLK = 640
NBLK = NPAD // BLK


def _make_edge_kernel():
  mesh = plsc.VectorSubcoreMesh(core_axis_name="c", subcore_axis_name="s")

  @functools.partial(
      pl.kernel,
      mesh=mesh,
      name="gat_edge_phase",
      compiler_params=pltpu.CompilerParams(use_tc_tiling_on_sc=False,
                                           needs_layout_passes=False),
      out_type=[
          jax.ShapeDtypeStruct((NC, NPAD, FH), jnp.float32),
          jax.ShapeDtypeStruct((NT, NPAD), jnp.float32),
      ],
      scratch_types=[
          pltpu.VMEM((2 * NPAD,), jnp.float32),
          pltpu.VMEM((4, 128), jnp.int32),
          pltpu.VMEM((4, 128), jnp.int32),
          pltpu.VMEM((128,), jnp.float32),
          pltpu.VMEM((NPAD,), jnp.float32),
          pltpu.VMEM((2, 128, FH), jnp.float32),
          pltpu.VMEM_SHARED((NPAD, FH), jnp.float32),
          pltpu.VMEM_SHARED((NPAD, FH), jnp.float32),
          pltpu.SemaphoreType.DMA,
          pltpu.SemaphoreType.DMA,
          pltpu.SemaphoreType.DMA,
          pltpu.SemaphoreType.DMA,
          pltpu.SemaphoreType.DMA,
          pltpu.SemaphoreType.DMA,
          pltpu.SemaphoreType.DMA,
          pltpu.SemaphoreType.DMA,
          pltpu.SemaphoreType.DMA,
          pltpu.SemaphoreType.DMA,
      ],
  )
  def edge_kernel(h_hbm, aa_hbm, src_hbm, dst_hbm, num_out, den_out,
                  aa_v, src_r, dst_r, ee_c, den_v, rows_v, h_spm, acc,
                  sem_g0, sem_g1, sem_i0, sem_i1, sem_i2, sem_i3,
                  sem_j0, sem_j1, sem_j2, sem_j3):
    c = lax.axis_index("c")
    s = lax.axis_index("s")
    wid = s * NC + c

    sem_g = (sem_g0, sem_g1)
    sem_i = (sem_i0, sem_i1, sem_i2, sem_i3)
    sem_j = (sem_j0, sem_j1, sem_j2, sem_j3)

    for q in range(4):
      pltpu.async_copy(src_hbm.at[s, q], src_r.at[q], sem_i[q])
      pltpu.async_copy(dst_hbm.at[s, q], dst_r.at[q], sem_j[q])
    pltpu.sync_copy(aa_hbm, aa_v)
    pltpu.sync_copy(h_hbm.at[c, pl.ds(s * RPS, RPS)],
                    h_spm.at[pl.ds(s * RPS, RPS)])

    zero16 = jnp.zeros((16,), jnp.float32)

    @pl.loop(0, NPAD // 16)
    def _(r):
      den_v[pl.ds(r * 16, 16)] = zero16

    @pl.loop(0, 128)
    def _(r):
      for dd in range(FH // 16):
        rows_v[0, r, pl.ds(dd * 16, 16)] = zero16

    for r0 in range(8):
      pltpu.sync_copy(rows_v.at[0, pl.ds(0, 80)],
                      acc.at[pl.ds(s * RPS + r0 * 80, 80)])
    plsc.subcore_barrier()

    def process(b2, b4):
      for kk in range(8):
        sl16 = pl.ds(kk * 16, 16)
        sv = src_r[b4, sl16]
        dv = dst_r[b4, sl16]
        e = (plsc.load_gather(aa_v, [sv * 2])
             + plsc.load_gather(aa_v, [dv * 2 + 1]))
        ee = jnp.exp(jnp.maximum(e, 0.2 * e))
        ee_c[sl16] = ee
        plsc.addupdate_scatter(den_v, [dv], ee)

      @pl.loop(0, 128, step=4)
      def _(j):
        for u in range(4):
          eev = plsc.load_gather(ee_c, [jnp.full((16,), 0, jnp.int32) + (j + u)])
          for dd in range(FH // 16):
            rows_v[b2, j + u, pl.ds(dd * 16, 16)] = (
                rows_v[b2, j + u, pl.ds(dd * 16, 16)] * eev)

      pltpu.sync_copy(rows_v.at[b2], acc.at[dst_r.at[b4]], add=True)

    pltpu.make_async_copy(src_hbm.at[s, 0], src_r.at[0], sem_i0).wait()
    pltpu.async_copy(h_spm.at[src_r.at[0]], rows_v.at[0], sem_g0)

    @pl.loop(0, NCH, step=4)
    def _(g):
      for b in range(4):
        gb = g + b
        b2 = b % 2
        bn = (b + 1) % 4
        nxt = jnp.minimum(gb + 1, NCH - 1)
        pltpu.make_async_copy(src_hbm.at[s, nxt], src_r.at[bn],
                              sem_i[bn]).wait()
        pltpu.async_copy(h_spm.at[src_r.at[bn]], rows_v.at[1 - b2],
                         sem_g[1 - b2])
        pltpu.make_async_copy(dst_hbm.at[s, nxt], dst_r.at[b],
                              sem_j[b]).wait()
        pltpu.make_async_copy(h_spm.at[src_r.at[b]], rows_v.at[b2],
                              sem_g[b2]).wait()
        process(b2, b)
        nxt4 = jnp.minimum(gb + 4, NCH - 1)
        pltpu.async_copy(src_hbm.at[s, nxt4], src_r.at[b], sem_i[b])
        pltpu.async_copy(dst_hbm.at[s, nxt4], dst_r.at[b], sem_j[b])

    pltpu.make_async_copy(h_spm.at[src_r.at[0]], rows_v.at[0],
                          sem_g0).wait()
    for q in range(1, 4):
      pltpu.make_async_copy(src_hbm.at[s, NCH - 1], src_r.at[q],
                            sem_i[q]).wait()
    for q in range(4):
      pltpu.make_async_copy(dst_hbm.at[s, NCH - 1], dst_r.at[q],
                            sem_j[q]).wait()

    pltpu.sync_copy(den_v, den_out.at[wid])
    plsc.subcore_barrier()

    for r0 in range(8):
      off = s * RPS + r0 * 80
      pltpu.sync_copy(acc.at[pl.ds(off, 80)], num_out.at[c, pl.ds(off, 80)])

  return edge_kernel


_edge128 = _make_edge_kernel()


def _rowmask(i, blk):
  row = i * blk + lax.broadcasted_iota(jnp.int32, (blk, 1), 0)
  return row < N


def _pre_body(x_ref, w_ref, a_ref, h_out, aa_out):
  h = jnp.dot(x_ref[...], w_ref[...], preferred_element_type=jnp.float32)
  h_out[0] = h[:, :FH]
  h_out[1] = h[:, FH:]
  aa_out[...] = jnp.dot(h, a_ref[...], preferred_element_type=jnp.float32)


_pre = pl.pallas_call(
    _pre_body,
    grid=(NBLK,),
    in_specs=[
        pl.BlockSpec((BLK, D_IN), lambda i: (i, 0)),
        pl.BlockSpec((D_IN, HID), lambda i: (0, 0)),
        pl.BlockSpec((HID, 2), lambda i: (0, 0)),
    ],
    out_specs=[
        pl.BlockSpec((NC, BLK, FH), lambda i: (0, i, 0)),
        pl.BlockSpec((BLK, 2), lambda i: (i, 0)),
    ],
    out_shape=[
        jax.ShapeDtypeStruct((NC, NPAD, FH), jnp.float32),
        jax.ShapeDtypeStruct((NPAD, 2), jnp.float32),
    ],
)


def _gat_from_acc(num_ref, den_ref, b):
  numer = jnp.concatenate([num_ref[0], num_ref[1]], axis=1)
  den = jnp.sum(den_ref[...].T, axis=1, keepdims=True) * 0.5
  return numer / (den + 1e-16) + b


def _pass1_body(num_ref, den_ref, b_ref, res_ref, fl_ref, x_out, part_out):
  i = pl.program_id(0)
  gat = _gat_from_acc(num_ref, den_ref, b_ref[...])
  gat = gat + res_ref[...] * fl_ref[0, 0]
  x = jnp.where(_rowmask(i, BLK), gat, 0.0)
  x_out[...] = x
  part_out[0, 0] = jnp.sum(x, axis=0)
  part_out[0, 1] = jnp.sum(x * x, axis=0)


_pass1 = pl.pallas_call(
    _pass1_body,
    grid=(NBLK,),
    in_specs=[
        pl.BlockSpec((NC, BLK, FH), lambda i: (0, i, 0)),
        pl.BlockSpec((NT, BLK), lambda i: (0, i)),
        pl.BlockSpec((1, HID), lambda i: (0, 0)),
        pl.BlockSpec((BLK, HID), lambda i: (i, 0)),
        pl.BlockSpec((1, 1), lambda i: (0, 0)),
    ],
    out_specs=[
        pl.BlockSpec((BLK, HID), lambda i: (i, 0)),
        pl.BlockSpec((1, 2, HID), lambda i: (i, 0, 0)),
    ],
    out_shape=[
        jax.ShapeDtypeStruct((NPAD, HID), jnp.float32),
        jax.ShapeDtypeStruct((NBLK, 2, HID), jnp.float32),
    ],
)


def _make_pass2(dout):
  def body(x_ref, part_ref, g_ref, bb_ref, w_ref, a_ref,
           h_out, aa_out, hn_out):
    i = pl.program_id(0)
    m = jnp.sum(part_ref[:, 0, :], axis=0) / N
    v = jnp.sum(part_ref[:, 1, :], axis=0) / N - m * m
    x = x_ref[...]
    xn = g_ref[...] * (x - m[None, :]) / jnp.sqrt(v + 1e-5)[None, :] + bb_ref[...]
    hn = jnp.maximum(xn, 0.1 * xn)
    hn = jnp.where(_rowmask(i, BLK), hn, 0.0)
    hn_out[...] = hn
    h2 = jnp.dot(hn, w_ref[...], preferred_element_type=jnp.float32)
    h_out[0] = h2[:, :FH]
    h_out[1] = h2[:, FH:]
    aa_out[...] = jnp.dot(h2, a_ref[...], preferred_element_type=jnp.float32)

  return pl.pallas_call(
      body,
      grid=(NBLK,),
      in_specs=[
          pl.BlockSpec((BLK, HID), lambda i: (i, 0)),
          pl.BlockSpec((NBLK, 2, HID), lambda i: (0, 0, 0)),
          pl.BlockSpec((1, HID), lambda i: (0, 0)),
          pl.BlockSpec((1, HID), lambda i: (0, 0)),
          pl.BlockSpec((HID, dout), lambda i: (0, 0)),
          pl.BlockSpec((dout, 2), lambda i: (0, 0)),
      ],
      out_specs=[
          pl.BlockSpec((NC, BLK, dout // 2), lambda i: (0, i, 0)),
          pl.BlockSpec((BLK, 2), lambda i: (i, 0)),
          pl.BlockSpec((BLK, HID), lambda i: (i, 0)),
      ],
      out_shape=[
          jax.ShapeDtypeStruct((NC, NPAD, dout // 2), jnp.float32),
          jax.ShapeDtypeStruct((NPAD, 2), jnp.float32),
          jax.ShapeDtypeStruct((NPAD, HID), jnp.float32),
      ],
  )


_pass2_128 = _make_pass2(HID)


def _final_body(x_ref, out_ref):
  g = x_ref[...]
  out_ref[...] = jnp.maximum(g, 0.1 * g)[:, :C]


_final = pl.pallas_call(
    _final_body,
    grid=(10,),
    in_specs=[pl.BlockSpec((1000, HID), lambda i: (i, 0))],
    out_specs=pl.BlockSpec((1000, C), lambda i: (i, 0)),
    out_shape=jax.ShapeDtypeStruct((N, C), jnp.float32),
)


def kernel(x, edge_index, edge_type, edge_distance, params):
  del edge_type, edge_distance
  sl = jnp.arange(N, dtype=jnp.int32)
  padv = jnp.full((EPAD - E2,), N, jnp.int32)
  src = jnp.concatenate([edge_index[0].astype(jnp.int32), sl, padv])
  dst = jnp.concatenate([edge_index[1].astype(jnp.int32), sl, padv])
  srcs = src.reshape(NS, NCH, 128)
  dsts = dst.reshape(NS, NCH, 128)

  xp = jnp.pad(x, ((0, NPAD - N), (0, 0)))

  def att(l):
    return jnp.stack([params[f"att_src{l}"], params[f"att_dst{l}"]], axis=1)

  def edge(h, aa):
    return _edge128(h, aa.reshape(-1), srcs, dsts)

  h, aa = _pre(xp, params["W0"], att(0))

  lf = N_MID + 1
  w4 = jnp.pad(params[f"W{lf}"], ((0, 0), (0, HID - C)))
  a4 = jnp.pad(att(lf), ((0, HID - C), (0, 0)))
  b4 = jnp.pad(params[f"b{lf}"], (0, HID - C))
  zvec = jnp.zeros((HID,), jnp.float32)
  bs = jnp.stack([params["b0"], params["b1"], params["b2"], params["b3"], b4]
                 )[:, None, :]
  bngs = jnp.stack([params["bn_g1"], params["bn_g2"], params["bn_g3"],
                    params["bn_g4"], jnp.ones((HID,), jnp.float32)])[:, None, :]
  bnbs = jnp.stack([params["bn_b1"], params["bn_b2"], params["bn_b3"],
                    params["bn_b4"], zvec])[:, None, :]
  ws = jnp.stack([params["W1"], params["W2"], params["W3"], w4,
                  jnp.zeros((HID, HID), jnp.float32)])
  aas = jnp.stack([att(1), att(2), att(3), a4, jnp.zeros((HID, 2), jnp.float32)])
  flags = jnp.array([0, 1, 1, 1, 0], jnp.float32).reshape(5, 1, 1)

  def body(carry, xs):
    hh, av, res, _ = carry
    b, g_, bb, w, a, fl = xs
    num, den = edge(hh, av)
    xt, part = _pass1(num, den, b, res, fl)
    h2, aa2, hn = _pass2_128(xt, part, g_, bb, w, a)
    return (h2, aa2, hn, xt), None

  zfeat = jnp.zeros((NPAD, HID), jnp.float32)
  (_, _, _, xt), _ = lax.scan(
      body, (h, aa, zfeat, zfeat), (bs, bngs, bnbs, ws, aas, flags))
  return _final(xt)
